# baseline (speedup 1.0000x reference)
import numpy as np
import jax
import jax.numpy as jnp
from jax.experimental import pallas as pl
from jax.experimental.pallas import tpu as pltpu

_N = 32
_TH = 1.1
_PAD_BEGIN = (_N - 1) // 2
_PAD_END = _N - 1 - _PAD_BEGIN
_C = 256
_BR = 32


def _band_matrices():
    mm = np.zeros((_C, _C), np.float32)
    mv = np.zeros((_C, _C), np.float32)
    mp = np.zeros((_C, _C), np.float32)
    for l in range(_C):
        lo = l - _PAD_BEGIN
        hi = l + _PAD_END
        for c in range(_C):
            if lo <= c - _C <= hi:
                mm[c, l] = 1.0
            if lo <= c <= hi:
                mv[c, l] = 1.0
            if lo <= c + _C <= hi:
                mp[c, l] = 1.0
    b = np.concatenate([mm, mv, mp], axis=0)
    return np.concatenate([b, b], axis=0)


_BANDS = _band_matrices()


def _hi16(a):
    u = pltpu.bitcast(a, jnp.uint32) & jnp.uint32(0xFFFF0000)
    return pltpu.bitcast(u, jnp.float32)


def _body(x_ref, m_ref, out_ref, sig_ref):
    T = x_ref.shape[-1]
    n_chunks = T // _C
    rows = _BR * n_chunks

    xv = jnp.concatenate(
        [x_ref[:, i * _C:(i + 1) * _C] for i in range(n_chunks)], axis=0)

    def banded(v):
        h = _hi16(v)
        l = v - h
        parts = []
        for a in (h, l):
            parts += [pltpu.roll(a, _BR, axis=0), a,
                      pltpu.roll(a, rows - _BR, axis=0)]
        y = jnp.concatenate(parts, axis=1)
        return jnp.dot(y, m_ref[...], preferred_element_type=jnp.float32)

    s1 = banded(xv)
    s2 = banded(xv * xv)

    inv_n = 1.0 / _N
    mean = s1 * inv_n
    var = jnp.maximum(s2 * inv_n - mean * mean, 0.0)
    sigma = jnp.sqrt(var)

    ridx = jax.lax.broadcasted_iota(jnp.int32, sigma.shape, 0)
    lane = jax.lax.broadcasted_iota(jnp.int32, sigma.shape, 1)
    bad = ((ridx < _BR) & (lane < _PAD_BEGIN)) | (
        (ridx >= rows - _BR) & (lane >= _C - _PAD_END))
    sigma = jnp.where(bad, 0.0, sigma)

    sigma_n = sigma * (1.0 / (2.0 * _TH))
    hard = jnp.clip(jnp.round(sigma_n), 0.0, 1.0)
    out = sigma_n + (hard - sigma_n)

    for i in range(n_chunks):
        sig_ref[:, i * _C:(i + 1) * _C] = sigma[i * _BR:(i + 1) * _BR, :]
        out_ref[:, i * _C:(i + 1) * _C] = out[i * _BR:(i + 1) * _BR, :]


@jax.jit
def kernel(input_attenuation):
    x = input_attenuation
    B, T = x.shape
    grid = (B // _BR,)
    spec = pl.BlockSpec((_BR, T), lambda i: (i, 0))
    bands = jnp.asarray(_BANDS)
    out, sig = pl.pallas_call(
        _body,
        grid=grid,
        in_specs=[spec, pl.BlockSpec((6 * _C, _C), lambda i: (0, 0))],
        out_specs=[spec, spec],
        out_shape=[jax.ShapeDtypeStruct((B, T), x.dtype)] * 2,
        compiler_params=pltpu.CompilerParams(
            dimension_semantics=("parallel",),
            vmem_limit_bytes=100 * 1024 * 1024,
        ),
        name="stdwet_dry_mxu",
    )(x, bands)
    return (out, sig)


# speedup vs baseline: 1.4297x; 1.4297x over previous
import numpy as np
import jax
import jax.numpy as jnp
from jax.experimental import pallas as pl
from jax.experimental.pallas import tpu as pltpu

_N = 32
_TH = 1.1
_PAD_BEGIN = (_N - 1) // 2
_PAD_END = _N - 1 - _PAD_BEGIN
_C = 256
_BR = 32


def _band_matrices():
    mm = np.zeros((_C, _C), np.float32)
    mv = np.zeros((_C, _C), np.float32)
    mp = np.zeros((_C, _C), np.float32)
    for l in range(_C):
        lo = l - _PAD_BEGIN
        hi = l + _PAD_END
        for c in range(_C):
            if lo <= c - _C <= hi:
                mm[c, l] = 1.0
            if lo <= c <= hi:
                mv[c, l] = 1.0
            if lo <= c + _C <= hi:
                mp[c, l] = 1.0
    bv = np.concatenate([mv, mv], axis=0)
    bm = np.concatenate([mm[_C // 2:, :_C // 2]] * 2, axis=0)
    bp = np.concatenate([mp[:_C // 2, _C // 2:]] * 2, axis=0)
    return bv, bm, bp


_BAND_V, _BAND_M, _BAND_P = _band_matrices()


def _hi16(a):
    u = pltpu.bitcast(a, jnp.uint32) & jnp.uint32(0xFFFF0000)
    return pltpu.bitcast(u, jnp.float32)


def _body(x_ref, mv_ref, mm_ref, mp_ref, out_ref, sig_ref):
    T = x_ref.shape[-1]
    n_chunks = T // _C
    rows = _BR * n_chunks
    half = _C // 2

    xv = jnp.concatenate(
        [x_ref[:, i * _C:(i + 1) * _C] for i in range(n_chunks)], axis=0)

    def banded(v):
        h = _hi16(v)
        l = v - h
        y = jnp.concatenate([h, l], axis=1)
        s = jnp.dot(y, mv_ref[...], preferred_element_type=jnp.float32)
        hm = pltpu.roll(h, _BR, axis=0)
        lm = pltpu.roll(l, _BR, axis=0)
        hp = pltpu.roll(h, rows - _BR, axis=0)
        lp = pltpu.roll(l, rows - _BR, axis=0)
        ym = jnp.concatenate([hm[:, half:], lm[:, half:]], axis=1)
        yp = jnp.concatenate([hp[:, :half], lp[:, :half]], axis=1)
        sm = jnp.dot(ym, mm_ref[...], preferred_element_type=jnp.float32)
        sp = jnp.dot(yp, mp_ref[...], preferred_element_type=jnp.float32)
        return s + jnp.concatenate([sm, sp], axis=1)

    s1 = banded(xv)
    s2 = banded(xv * xv)

    inv_n = 1.0 / _N
    mean = s1 * inv_n
    var = jnp.maximum(s2 * inv_n - mean * mean, 0.0)
    sigma = jnp.sqrt(var)

    ridx = jax.lax.broadcasted_iota(jnp.int32, sigma.shape, 0)
    lane = jax.lax.broadcasted_iota(jnp.int32, sigma.shape, 1)
    bad = ((ridx < _BR) & (lane < _PAD_BEGIN)) | (
        (ridx >= rows - _BR) & (lane >= _C - _PAD_END))
    sigma = jnp.where(bad, 0.0, sigma)

    sigma_n = sigma * (1.0 / (2.0 * _TH))
    hard = jnp.clip(jnp.round(sigma_n), 0.0, 1.0)
    out = sigma_n + (hard - sigma_n)

    for i in range(n_chunks):
        sig_ref[:, i * _C:(i + 1) * _C] = sigma[i * _BR:(i + 1) * _BR, :]
        out_ref[:, i * _C:(i + 1) * _C] = out[i * _BR:(i + 1) * _BR, :]


@jax.jit
def kernel(input_attenuation):
    x = input_attenuation
    B, T = x.shape
    grid = (B // _BR,)
    spec = pl.BlockSpec((_BR, T), lambda i: (i, 0))
    const = lambda shape: pl.BlockSpec(shape, lambda i: (0, 0))
    out, sig = pl.pallas_call(
        _body,
        grid=grid,
        in_specs=[spec,
                  const((2 * _C, _C)),
                  const((_C, _C // 2)),
                  const((_C, _C // 2))],
        out_specs=[spec, spec],
        out_shape=[jax.ShapeDtypeStruct((B, T), x.dtype)] * 2,
        compiler_params=pltpu.CompilerParams(
            dimension_semantics=("parallel",),
            vmem_limit_bytes=100 * 1024 * 1024,
        ),
        name="stdwet_dry_mxu",
    )(x, jnp.asarray(_BAND_V), jnp.asarray(_BAND_M), jnp.asarray(_BAND_P))
    return (out, sig)


# speedup vs baseline: 1.4568x; 1.0190x over previous
import numpy as np
import jax
import jax.numpy as jnp
from jax.experimental import pallas as pl
from jax.experimental.pallas import tpu as pltpu

_N = 32
_TH = 1.1
_PAD_BEGIN = (_N - 1) // 2
_PAD_END = _N - 1 - _PAD_BEGIN
_C = 256
_BR = 32


def _band_matrices():
    mm = np.zeros((_C, _C), np.float32)
    mv = np.zeros((_C, _C), np.float32)
    mp = np.zeros((_C, _C), np.float32)
    for l in range(_C):
        lo = l - _PAD_BEGIN
        hi = l + _PAD_END
        for c in range(_C):
            if lo <= c - _C <= hi:
                mm[c, l] = 1.0
            if lo <= c <= hi:
                mv[c, l] = 1.0
            if lo <= c + _C <= hi:
                mp[c, l] = 1.0
    bv = np.concatenate([mv, mv], axis=0)
    bm = np.concatenate([mm[_C // 2:, :_C // 2]] * 2, axis=0)
    bp = np.concatenate([mp[:_C // 2, _C // 2:]] * 2, axis=0)
    return bv, bm, bp


_BAND_V, _BAND_M, _BAND_P = _band_matrices()


def _hi16(a):
    u = pltpu.bitcast(a, jnp.uint32) & jnp.uint32(0xFFFF0000)
    return pltpu.bitcast(u, jnp.float32)


def _body(x_ref, mv_ref, mm_ref, mp_ref, out_ref, sig_ref):
    T = x_ref.shape[-1]
    n_chunks = T // _C
    rows = _BR * n_chunks
    half = _C // 2

    xv = jnp.concatenate(
        [x_ref[:, i * _C:(i + 1) * _C] for i in range(n_chunks)], axis=0)

    def banded(v):
        h = _hi16(v)
        l = v - h
        y = jnp.concatenate([h, l], axis=1)
        s = jnp.dot(y, mv_ref[...], preferred_element_type=jnp.float32)
        hm = pltpu.roll(h, _BR, axis=0)
        lm = pltpu.roll(l, _BR, axis=0)
        hp = pltpu.roll(h, rows - _BR, axis=0)
        lp = pltpu.roll(l, rows - _BR, axis=0)
        ym = jnp.concatenate([hm[:, half:], lm[:, half:]], axis=1)
        yp = jnp.concatenate([hp[:, :half], lp[:, :half]], axis=1)
        sm = jnp.dot(ym, mm_ref[...], preferred_element_type=jnp.float32)
        sp = jnp.dot(yp, mp_ref[...], preferred_element_type=jnp.float32)
        return s + jnp.concatenate([sm, sp], axis=1)

    s1 = banded(xv)
    s2 = banded(xv * xv)

    inv_n = 1.0 / _N
    mean = s1 * inv_n
    d = s2 * inv_n - mean * mean
    var = jnp.where(d > 0.0, d, 0.0)
    sigma = jnp.sqrt(var)

    lane = jax.lax.broadcasted_iota(jnp.int32, (_BR, _C), 1)
    first = jnp.where(lane < _PAD_BEGIN, 0.0, sigma[0:_BR, :])
    last = jnp.where(lane >= _C - _PAD_END, 0.0, sigma[rows - _BR:rows, :])
    sigma = jnp.concatenate([first, sigma[_BR:rows - _BR, :], last], axis=0)

    sigma_n = sigma * (1.0 / (2.0 * _TH))
    out = jnp.clip(jnp.round(sigma_n), 0.0, 1.0)

    for i in range(n_chunks):
        sig_ref[:, i * _C:(i + 1) * _C] = sigma[i * _BR:(i + 1) * _BR, :]
        out_ref[:, i * _C:(i + 1) * _C] = out[i * _BR:(i + 1) * _BR, :]


@jax.jit
def kernel(input_attenuation):
    x = input_attenuation
    B, T = x.shape
    grid = (B // _BR,)
    spec = pl.BlockSpec((_BR, T), lambda i: (i, 0))
    const = lambda shape: pl.BlockSpec(shape, lambda i: (0, 0))
    out, sig = pl.pallas_call(
        _body,
        grid=grid,
        in_specs=[spec,
                  const((2 * _C, _C)),
                  const((_C, _C // 2)),
                  const((_C, _C // 2))],
        out_specs=[spec, spec],
        out_shape=[jax.ShapeDtypeStruct((B, T), x.dtype)] * 2,
        compiler_params=pltpu.CompilerParams(
            dimension_semantics=("parallel",),
            vmem_limit_bytes=100 * 1024 * 1024,
        ),
        name="stdwet_dry_mxu",
    )(x, jnp.asarray(_BAND_V), jnp.asarray(_BAND_M), jnp.asarray(_BAND_P))
    return (out, sig)


# speedup vs baseline: 1.4817x; 1.0171x over previous
import numpy as np
import jax
import jax.numpy as jnp
from jax.experimental import pallas as pl
from jax.experimental.pallas import tpu as pltpu

_N = 32
_TH = 1.1
_PAD_BEGIN = (_N - 1) // 2
_PAD_END = _N - 1 - _PAD_BEGIN
_C = 256
_BR = 64


def _band_matrices():
    mm = np.zeros((_C, _C), np.float32)
    mv = np.zeros((_C, _C), np.float32)
    mp = np.zeros((_C, _C), np.float32)
    for l in range(_C):
        lo = l - _PAD_BEGIN
        hi = l + _PAD_END
        for c in range(_C):
            if lo <= c - _C <= hi:
                mm[c, l] = 1.0
            if lo <= c <= hi:
                mv[c, l] = 1.0
            if lo <= c + _C <= hi:
                mp[c, l] = 1.0
    bv = np.concatenate([mv, mv], axis=0)
    bm = np.concatenate([mm[_C // 2:, :_C // 2]] * 2, axis=0)
    bp = np.concatenate([mp[:_C // 2, _C // 2:]] * 2, axis=0)
    return bv, bm, bp


_BAND_V, _BAND_M, _BAND_P = _band_matrices()


def _hi16(a):
    u = pltpu.bitcast(a, jnp.uint32) & jnp.uint32(0xFFFF0000)
    return pltpu.bitcast(u, jnp.float32)


def _body(x_ref, mv_ref, mm_ref, mp_ref, out_ref, sig_ref):
    T = x_ref.shape[-1]
    n_chunks = T // _C
    rows = _BR * n_chunks
    half = _C // 2

    xv = jnp.concatenate(
        [x_ref[:, i * _C:(i + 1) * _C] for i in range(n_chunks)], axis=0)

    def banded(v):
        h = _hi16(v)
        l = v - h
        y = jnp.concatenate([h, l], axis=1)
        s = jnp.dot(y, mv_ref[...], preferred_element_type=jnp.float32)
        hm = pltpu.roll(h, _BR, axis=0)
        lm = pltpu.roll(l, _BR, axis=0)
        hp = pltpu.roll(h, rows - _BR, axis=0)
        lp = pltpu.roll(l, rows - _BR, axis=0)
        ym = jnp.concatenate([hm[:, half:], lm[:, half:]], axis=1)
        yp = jnp.concatenate([hp[:, :half], lp[:, :half]], axis=1)
        sm = jnp.dot(ym, mm_ref[...], preferred_element_type=jnp.float32)
        sp = jnp.dot(yp, mp_ref[...], preferred_element_type=jnp.float32)
        return s + jnp.concatenate([sm, sp], axis=1)

    s1 = banded(xv)
    s2 = banded(xv * xv)

    d = jnp.float32(_N) * s2 - s1 * s1
    var_n2 = jnp.where(d > 0.0, d, 0.0)
    rt = jnp.sqrt(var_n2)
    sigma = rt * (1.0 / _N)

    lane = jax.lax.broadcasted_iota(jnp.int32, (_BR, _C), 1)
    first = jnp.where(lane < _PAD_BEGIN, 0.0, sigma[0:_BR, :])
    last = jnp.where(lane >= _C - _PAD_END, 0.0, sigma[rows - _BR:rows, :])
    sigma = jnp.concatenate([first, sigma[_BR:rows - _BR, :], last], axis=0)

    out = jnp.where(sigma > jnp.float32(_TH), 1.0, 0.0)

    for i in range(n_chunks):
        sig_ref[:, i * _C:(i + 1) * _C] = sigma[i * _BR:(i + 1) * _BR, :]
        out_ref[:, i * _C:(i + 1) * _C] = out[i * _BR:(i + 1) * _BR, :]


@jax.jit
def kernel(input_attenuation):
    x = input_attenuation
    B, T = x.shape
    grid = (B // _BR,)
    spec = pl.BlockSpec((_BR, T), lambda i: (i, 0))
    const = lambda shape: pl.BlockSpec(shape, lambda i: (0, 0))
    out, sig = pl.pallas_call(
        _body,
        grid=grid,
        in_specs=[spec,
                  const((2 * _C, _C)),
                  const((_C, _C // 2)),
                  const((_C, _C // 2))],
        out_specs=[spec, spec],
        out_shape=[jax.ShapeDtypeStruct((B, T), x.dtype)] * 2,
        compiler_params=pltpu.CompilerParams(
            dimension_semantics=("parallel",),
            vmem_limit_bytes=100 * 1024 * 1024,
        ),
        name="stdwet_dry_mxu",
    )(x, jnp.asarray(_BAND_V), jnp.asarray(_BAND_M), jnp.asarray(_BAND_P))
    return (out, sig)
